# Initial kernel scaffold; baseline (speedup 1.0000x reference)
import functools

import jax
import jax.numpy as jnp
from jax.experimental import pallas as pl
from jax.experimental.pallas import tpu as pltpu


def _pick_block(n, candidates):
    for c in candidates:
        if n % c == 0:
            return c
    return n


def _gcn_body(adj_ref, mask_ref, x_ref, xi_ref, w_ref, out_ref, acc_ref):
    j = pl.program_id(1)

    @pl.when(j == 0)
    def _init():
        acc_ref[...] = xi_ref[...]

    a = adj_ref[...] * mask_ref[...]
    acc_ref[...] += jnp.dot(a, x_ref[...], preferred_element_type=jnp.float32)

    @pl.when(j == pl.num_programs(1) - 1)
    def _finish():
        out_ref[...] = jnp.dot(
            acc_ref[...], w_ref[...].T, preferred_element_type=jnp.float32
        )


@jax.jit
def kernel(x, adj, mask, W):
    n, c_in = x.shape
    c_out = W.shape[0]

    bm = _pick_block(n, (2000, 1000, 2500, 500, 250, 125, 8))
    bk = _pick_block(n, (500, 1000, 250, 2000, 125, 8))

    grid = (n // bm, n // bk)

    return pl.pallas_call(
        _gcn_body,
        grid=grid,
        in_specs=[
            pl.BlockSpec((bm, bk), lambda i, j: (i, j)),
            pl.BlockSpec((bm, bk), lambda i, j: (i, j)),
            pl.BlockSpec((bk, c_in), lambda i, j: (j, 0)),
            pl.BlockSpec((bm, c_in), lambda i, j: (i, 0)),
            pl.BlockSpec((c_out, c_in), lambda i, j: (0, 0)),
        ],
        out_specs=pl.BlockSpec((bm, c_out), lambda i, j: (i, 0)),
        out_shape=jax.ShapeDtypeStruct((n, c_out), jnp.float32),
        scratch_shapes=[pltpu.VMEM((bm, c_in), jnp.float32)],
        compiler_params=pltpu.CompilerParams(
            dimension_semantics=("parallel", "arbitrary"),
        ),
    )(adj, mask, x, x, W)


# speedup vs baseline: 1.0016x; 1.0016x over previous
import jax
import jax.numpy as jnp
from jax.experimental import pallas as pl
from jax.experimental.pallas import tpu as pltpu


def _pick_block(n, candidates):
    for c in candidates:
        if n % c == 0:
            return c
    return n


def _gcn_body(adj_ref, mask_ref, x_ref, xi_ref, w_ref, out_ref):
    a = adj_ref[...] * mask_ref[...]
    s = jnp.dot(a, x_ref[...], preferred_element_type=jnp.float32) + xi_ref[...]
    out_ref[...] = jnp.dot(s, w_ref[...].T, preferred_element_type=jnp.float32)


@jax.jit
def kernel(x, adj, mask, W):
    n, c_in = x.shape
    c_out = W.shape[0]

    bm = _pick_block(n, (200, 80, 40, 16, 8))
    grid = (n // bm,)

    return pl.pallas_call(
        _gcn_body,
        grid=grid,
        in_specs=[
            pl.BlockSpec((bm, n), lambda i: (i, 0)),
            pl.BlockSpec((bm, n), lambda i: (i, 0)),
            pl.BlockSpec((n, c_in), lambda i: (0, 0)),
            pl.BlockSpec((bm, c_in), lambda i: (i, 0)),
            pl.BlockSpec((c_out, c_in), lambda i: (0, 0)),
        ],
        out_specs=pl.BlockSpec((bm, c_out), lambda i: (i, 0)),
        out_shape=jax.ShapeDtypeStruct((n, c_out), jnp.float32),
        compiler_params=pltpu.CompilerParams(
            dimension_semantics=("parallel",),
        ),
    )(adj, mask, x, x, W)
